# baseline (speedup 1.0000x reference)
import jax
import jax.numpy as jnp
from jax.experimental import pallas as pl
from jax.experimental.pallas import tpu as pltpu

_D_MODEL = 192
_NUM_FRAMES = 3
_EMB_SCALE = 0.001
_PE_SCALE = 0.0001
_CB = 16
_GROUPS = _CB // 4


def _fill_group(idv_ref, pe_ref, k, g, H, W):
    inv = idv_ref[k]
    wpos = jax.lax.broadcasted_iota(jnp.int32, (H, W), 1).astype(jnp.float32)
    hpos = jax.lax.broadcasted_iota(jnp.int32, (H, W), 0).astype(jnp.float32)
    phx = wpos * inv
    phy = hpos * inv
    pe_ref[4 * g + 0] = jnp.sin(phx) * _PE_SCALE
    pe_ref[4 * g + 1] = jnp.cos(phx) * _PE_SCALE
    pe_ref[4 * g + 2] = jnp.sin(phy) * _PE_SCALE
    pe_ref[4 * g + 3] = jnp.cos(phy) * _PE_SCALE


def _make_add_kernel(n_cb, H, W):
    def _add_kernel(fn_ref, idv_ref, x_ref, ft_ref, o_ref, pe_a, pe_b):
        c = pl.program_id(0)
        b = pl.program_id(1)
        even = jax.lax.rem(c, 2) == 0

        @pl.when(jnp.logical_and(c == 0, b == 0))
        def _():
            for g in range(_GROUPS):
                _fill_group(idv_ref, pe_a, g, g, H, W)

        for g in range(_GROUPS):
            cond = jnp.logical_and(b == g + 1, c + 1 < n_cb)

            @pl.when(jnp.logical_and(cond, jnp.logical_not(even)))
            def _(g=g):
                _fill_group(idv_ref, pe_a, (c + 1) * _GROUPS + g, g, H, W)

            @pl.when(jnp.logical_and(cond, even))
            def _(g=g):
                _fill_group(idv_ref, pe_b, (c + 1) * _GROUPS + g, g, H, W)

        fn = fn_ref[b]
        ft = ft_ref[0]
        rows = jax.lax.broadcasted_iota(jnp.int32, (_NUM_FRAMES, _CB), 0)
        femb = jnp.sum(jnp.where(rows == fn, ft, 0.0), axis=0)
        add = (femb * _EMB_SCALE)[None, :, None, None]

        @pl.when(even)
        def _():
            o_ref[...] = x_ref[...] + pe_a[...][None] + add

        @pl.when(jnp.logical_not(even))
        def _():
            o_ref[...] = x_ref[...] + pe_b[...][None] + add

    return _add_kernel


def kernel(x, frame_number, frame_table, pe):
    B, C, H, W = x.shape
    n_cb = C // _CB
    ft3 = jnp.transpose(
        jnp.reshape(frame_table, (_NUM_FRAMES, n_cb, _CB)), (1, 0, 2)
    )
    fn = frame_number.astype(jnp.int32)
    k = jnp.arange(C // 4, dtype=jnp.float32)
    inv_div = jnp.exp(-jnp.log(10000.0) * k * (4.0 / C)).astype(jnp.float32)

    grid_spec = pltpu.PrefetchScalarGridSpec(
        num_scalar_prefetch=2,
        grid=(n_cb, B),
        in_specs=[
            pl.BlockSpec((1, _CB, H, W), lambda c, b, *_: (b, c, 0, 0)),
            pl.BlockSpec((1, _NUM_FRAMES, _CB), lambda c, b, *_: (c, 0, 0)),
        ],
        out_specs=pl.BlockSpec((1, _CB, H, W), lambda c, b, *_: (b, c, 0, 0)),
        scratch_shapes=[
            pltpu.VMEM((_CB, H, W), jnp.float32),
            pltpu.VMEM((_CB, H, W), jnp.float32),
        ],
    )
    return pl.pallas_call(
        _make_add_kernel(n_cb, H, W),
        grid_spec=grid_spec,
        out_shape=jax.ShapeDtypeStruct(x.shape, x.dtype),
        compiler_params=pltpu.CompilerParams(
            dimension_semantics=("arbitrary", "arbitrary"),
        ),
    )(fn, inv_div, x, ft3)


# speedup vs baseline: 1.0377x; 1.0377x over previous
import jax
import jax.numpy as jnp
from jax.experimental import pallas as pl
from jax.experimental.pallas import tpu as pltpu
from jax._src.pallas.core import RevisitMode as _RevisitMode

_D_MODEL = 192
_NUM_FRAMES = 3
_EMB_SCALE = 0.001
_PE_SCALE = 0.0001
_CB = 32
_GROUPS = _CB // 4


def _fill_group(idv_ref, pe_ref, k, g, H, W):
    inv = idv_ref[k]
    wpos = jax.lax.broadcasted_iota(jnp.int32, (H, W), 1).astype(jnp.float32)
    hpos = jax.lax.broadcasted_iota(jnp.int32, (H, W), 0).astype(jnp.float32)
    phx = wpos * inv
    phy = hpos * inv
    pe_ref[4 * g + 0] = jnp.sin(phx) * _PE_SCALE
    pe_ref[4 * g + 1] = jnp.cos(phx) * _PE_SCALE
    pe_ref[4 * g + 2] = jnp.sin(phy) * _PE_SCALE
    pe_ref[4 * g + 3] = jnp.cos(phy) * _PE_SCALE


def _make_add_kernel(n_cb, H, W):
    def _add_kernel(fn_ref, idv_ref, x_ref, ft_ref, pe0_ref, o_ref, pe_a, pe_b):
        c = pl.program_id(0)
        b = pl.program_id(1)
        even = jax.lax.rem(c, 2) == 0

        for g in range(_GROUPS):
            cond = jnp.logical_and(b == g + 1, c + 1 < n_cb)

            @pl.when(jnp.logical_and(cond, jnp.logical_not(even)))
            def _(g=g):
                _fill_group(idv_ref, pe_a, (c + 1) * _GROUPS + g, g, H, W)

            @pl.when(jnp.logical_and(cond, even))
            def _(g=g):
                _fill_group(idv_ref, pe_b, (c + 1) * _GROUPS + g, g, H, W)

        fn = fn_ref[b]
        ft = ft_ref[0]
        rows = jax.lax.broadcasted_iota(jnp.int32, (_NUM_FRAMES, _CB), 0)
        femb = jnp.sum(jnp.where(rows == fn, ft, 0.0), axis=0)
        add = (femb * _EMB_SCALE)[None, :, None, None]

        @pl.when(c == 0)
        def _():
            o_ref[...] = x_ref[...] + pe0_ref[...][None] + add

        @pl.when(jnp.logical_and(c != 0, even))
        def _():
            o_ref[...] = x_ref[...] + pe_a[...][None] + add

        @pl.when(jnp.logical_and(c != 0, jnp.logical_not(even)))
        def _():
            o_ref[...] = x_ref[...] + pe_b[...][None] + add

    return _add_kernel


def kernel(x, frame_number, frame_table, pe):
    B, C, H, W = x.shape
    n_cb = C // _CB
    ft3 = jnp.transpose(
        jnp.reshape(frame_table, (_NUM_FRAMES, n_cb, _CB)), (1, 0, 2)
    )
    fn = frame_number.astype(jnp.int32)
    k = jnp.arange(C // 4, dtype=jnp.float32)
    inv_div = jnp.exp(-jnp.log(10000.0) * k * (4.0 / C)).astype(jnp.float32)

    grid_spec = pltpu.PrefetchScalarGridSpec(
        num_scalar_prefetch=2,
        grid=(n_cb, B),
        in_specs=[
            pl.BlockSpec((1, _CB, H, W), lambda c, b, *_: (b, c, 0, 0)),
            pl.BlockSpec((1, _NUM_FRAMES, _CB), lambda c, b, *_: (c, 0, 0)),
            pl.BlockSpec(
                (_CB, H, W),
                lambda c, b, *_: (0, 0, 0),
                pipeline_mode=pl.Buffered(
                    buffer_count=1, revisit=_RevisitMode.ANY
                ),
            ),
        ],
        out_specs=pl.BlockSpec((1, _CB, H, W), lambda c, b, *_: (b, c, 0, 0)),
        scratch_shapes=[
            pltpu.VMEM((_CB, H, W), jnp.float32),
            pltpu.VMEM((_CB, H, W), jnp.float32),
        ],
    )
    return pl.pallas_call(
        _make_add_kernel(n_cb, H, W),
        grid_spec=grid_spec,
        out_shape=jax.ShapeDtypeStruct(x.shape, x.dtype),
        compiler_params=pltpu.CompilerParams(
            dimension_semantics=("arbitrary", "arbitrary"),
        ),
    )(fn, inv_div, x, ft3, pe)


# speedup vs baseline: 1.0377x; 1.0000x over previous
import jax
import jax.numpy as jnp
from jax.experimental import pallas as pl
from jax.experimental.pallas import tpu as pltpu
from jax._src.pallas.core import RevisitMode as _RevisitMode

_D_MODEL = 192
_NUM_FRAMES = 3
_EMB_SCALE = 0.001
_PE_SCALE = 0.0001
_CB = 32
_GROUPS = _CB // 4


def _fill_group(idv_ref, pe_ref, k, g, H, W):
    inv = idv_ref[k]
    wpos = jax.lax.broadcasted_iota(jnp.int32, (H, W), 1).astype(jnp.float32)
    hpos = jax.lax.broadcasted_iota(jnp.int32, (H, W), 0).astype(jnp.float32)
    phx = wpos * inv
    phy = hpos * inv
    pe_ref[4 * g + 0] = (jnp.sin(phx) * _PE_SCALE).astype(jnp.bfloat16)
    pe_ref[4 * g + 1] = (jnp.cos(phx) * _PE_SCALE).astype(jnp.bfloat16)
    pe_ref[4 * g + 2] = (jnp.sin(phy) * _PE_SCALE).astype(jnp.bfloat16)
    pe_ref[4 * g + 3] = (jnp.cos(phy) * _PE_SCALE).astype(jnp.bfloat16)


def _make_add_kernel(n_cb, H, W):
    def _add_kernel(fn_ref, idv_ref, x_ref, ft_ref, pe0_ref, o_ref, pe_a, pe_b):
        c = pl.program_id(0)
        b = pl.program_id(1)
        even = jax.lax.rem(c, 2) == 0

        for g in range(_GROUPS):
            cond = jnp.logical_and(b == g + 1, c + 1 < n_cb)

            @pl.when(jnp.logical_and(cond, jnp.logical_not(even)))
            def _(g=g):
                _fill_group(idv_ref, pe_a, (c + 1) * _GROUPS + g, g, H, W)

            @pl.when(jnp.logical_and(cond, even))
            def _(g=g):
                _fill_group(idv_ref, pe_b, (c + 1) * _GROUPS + g, g, H, W)

        fn = fn_ref[b]
        ft = ft_ref[0]
        rows = jax.lax.broadcasted_iota(jnp.int32, (_NUM_FRAMES, _CB), 0)
        femb = jnp.sum(jnp.where(rows == fn, ft, 0.0), axis=0)
        add = (femb * _EMB_SCALE)[None, :, None, None]

        @pl.when(c == 0)
        def _():
            o_ref[...] = x_ref[...] + pe0_ref[...][None] + add

        @pl.when(jnp.logical_and(c != 0, even))
        def _():
            o_ref[...] = x_ref[...] + pe_a[...].astype(jnp.float32)[None] + add

        @pl.when(jnp.logical_and(c != 0, jnp.logical_not(even)))
        def _():
            o_ref[...] = x_ref[...] + pe_b[...].astype(jnp.float32)[None] + add

    return _add_kernel


def kernel(x, frame_number, frame_table, pe):
    B, C, H, W = x.shape
    n_cb = C // _CB
    ft3 = jnp.transpose(
        jnp.reshape(frame_table, (_NUM_FRAMES, n_cb, _CB)), (1, 0, 2)
    )
    fn = frame_number.astype(jnp.int32)
    k = jnp.arange(C // 4, dtype=jnp.float32)
    inv_div = jnp.exp(-jnp.log(10000.0) * k * (4.0 / C)).astype(jnp.float32)

    grid_spec = pltpu.PrefetchScalarGridSpec(
        num_scalar_prefetch=2,
        grid=(n_cb, B),
        in_specs=[
            pl.BlockSpec((1, _CB, H, W), lambda c, b, *_: (b, c, 0, 0)),
            pl.BlockSpec((1, _NUM_FRAMES, _CB), lambda c, b, *_: (c, 0, 0)),
            pl.BlockSpec(
                (_CB, H, W),
                lambda c, b, *_: (0, 0, 0),
                pipeline_mode=pl.Buffered(
                    buffer_count=1, revisit=_RevisitMode.ANY
                ),
            ),
        ],
        out_specs=pl.BlockSpec((1, _CB, H, W), lambda c, b, *_: (b, c, 0, 0)),
        scratch_shapes=[
            pltpu.VMEM((_CB, H, W), jnp.bfloat16),
            pltpu.VMEM((_CB, H, W), jnp.bfloat16),
        ],
    )
    return pl.pallas_call(
        _make_add_kernel(n_cb, H, W),
        grid_spec=grid_spec,
        out_shape=jax.ShapeDtypeStruct(x.shape, x.dtype),
        compiler_params=pltpu.CompilerParams(
            dimension_semantics=("arbitrary", "arbitrary"),
        ),
    )(fn, inv_div, x, ft3, pe)
